# Initial kernel scaffold; baseline (speedup 1.0000x reference)
import jax
import jax.numpy as jnp
from jax.experimental import pallas as pl


def kernel(features, anchors, rpn_delta, rpn_targets, W1, b1, W2, b2, Wd, bd, Wt, bt, img_h, img_w):
    raise NotImplementedError("write your pallas kernel here")


# speedup vs baseline: 1.2401x; 1.2401x over previous
import functools

import jax
import jax.numpy as jnp
from jax import lax
from jax.experimental import pallas as pl
from jax.experimental.pallas import tpu as pltpu

_P = 7
_PRE = 6000
_POST = 300
_NPAD = 304
_NCLS = 21


def _decode_boxes(anchors, deltas):
    wa = anchors[:, 2] - anchors[:, 0]
    ha = anchors[:, 3] - anchors[:, 1]
    cxa = anchors[:, 0] + 0.5 * wa
    cya = anchors[:, 1] + 0.5 * ha
    cx = deltas[:, 0] * wa + cxa
    cy = deltas[:, 1] * ha + cya
    w = jnp.exp(deltas[:, 2]) * wa
    h = jnp.exp(deltas[:, 3]) * ha
    return jnp.stack([cx - 0.5 * w, cy - 0.5 * h, cx + 0.5 * w, cy + 0.5 * h], axis=1)


def _nms_f64(boxes, scores, iou_thr, max_keep):
    n = boxes.shape[0]
    areas = (boxes[:, 2] - boxes[:, 0]) * (boxes[:, 3] - boxes[:, 1])
    idxs = jnp.arange(n)

    def body(i, state):
        active, keep = state
        sc = jnp.where(active, scores, -jnp.inf)
        best = jnp.argmax(sc)
        has = sc[best] > -jnp.inf
        xx1 = jnp.maximum(boxes[best, 0], boxes[:, 0])
        yy1 = jnp.maximum(boxes[best, 1], boxes[:, 1])
        xx2 = jnp.minimum(boxes[best, 2], boxes[:, 2])
        yy2 = jnp.minimum(boxes[best, 3], boxes[:, 3])
        inter = jnp.maximum(xx2 - xx1, 0.0) * jnp.maximum(yy2 - yy1, 0.0)
        iou = inter / jnp.maximum(areas[best] + areas - inter, 1e-9)
        active = active & (iou <= iou_thr) & (idxs != best)
        keep = keep.at[i].set(jnp.where(has, best.astype(keep.dtype), keep[0]))
        return (active, keep)

    active0 = scores > -jnp.inf
    keep0 = jnp.zeros((max_keep,), dtype=jnp.int32)
    _, keep = jax.lax.fori_loop(0, max_keep, body, (active0, keep0))
    return keep


def _make_rois(anchors, rpn_delta, rpn_targets, img_h, img_w):
    boxes = _decode_boxes(anchors, rpn_delta[0])
    w_lim = jnp.asarray(img_w).astype(boxes.dtype)
    h_lim = jnp.asarray(img_h).astype(boxes.dtype)
    boxes = jnp.stack(
        [
            jnp.clip(boxes[:, 0], 0.0, w_lim),
            jnp.clip(boxes[:, 1], 0.0, h_lim),
            jnp.clip(boxes[:, 2], 0.0, w_lim),
            jnp.clip(boxes[:, 3], 0.0, h_lim),
        ],
        axis=1,
    )
    keep = ((boxes[:, 2] - boxes[:, 0]) >= 16.0) & ((boxes[:, 3] - boxes[:, 1]) >= 16.0)
    scores = rpn_targets[0].reshape(-1, 2)[:, 1]
    masked = jnp.where(keep, scores, -jnp.inf)
    s, top = jax.lax.top_k(masked, _PRE)
    b = boxes[top].astype(jnp.float64)
    s = s.astype(jnp.float64)
    k = _nms_f64(b, s, 0.7, _POST)
    return b[k].astype(jnp.float32)


def _bin_bounds(off, length):
    q = jnp.arange(_P, dtype=jnp.int32)[None, :]
    length = length[:, None]
    starts = (q * length) // _P
    ends = jnp.minimum(jnp.maximum(((q + 1) * length) // _P, starts + 1), length)
    ends = jnp.maximum(ends, starts + 1)
    return off[:, None] + starts, off[:, None] + ends


def _floor_log2(x):
    k = jnp.zeros_like(x)
    for t in (2, 4, 8, 16, 32):
        k = k + (x >= t).astype(x.dtype)
    return k


def _roipool_body(ky_ref, gy1_ref, gy2_ref, xs_ref, xe_ref, feat_ref, out_ref,
                  tab_ref, *, fh, fw, ch):
    n = pl.program_id(0)

    @pl.when(n == 0)
    def _build():
        tab_ref[0] = feat_ref[...]
        for k in range(1, 6):
            s = 1 << (k - 1)
            tab_ref[k, : fh - s] = jnp.maximum(
                tab_ref[k - 1, : fh - s], tab_ref[k - 1, s:])
            last = tab_ref[k - 1, fh - 1: fh]
            tab_ref[k, fh - s:] = jnp.maximum(
                tab_ref[k - 1, fh - s:], jnp.broadcast_to(last, (s, fw, ch)))

    xpos = lax.broadcasted_iota(jnp.int32, (fw, ch), 0)
    for qy in range(_P):
        k = ky_ref[n, qy]
        a = gy1_ref[n, qy]
        b2 = gy2_ref[n, qy]
        row1 = tab_ref[pl.ds(k, 1), pl.ds(a, 1)][0, 0]
        row2 = tab_ref[pl.ds(k, 1), pl.ds(b2, 1)][0, 0]
        rp = jnp.maximum(row1, row2)
        for qx in range(_P):
            m = (xpos >= xs_ref[n, qx]) & (xpos < xe_ref[n, qx])
            col = jnp.max(jnp.where(m, rp, -jnp.inf), axis=0)
            out_ref[qy * _P + qx, 0, 0, :] = col


def _roi_pool_pallas(featT, roi):
    fh, fw, ch = featT.shape
    scale = 1.0 / 16.0
    x1 = jnp.floor(roi[:, 0] * scale).astype(jnp.int32)
    y1 = jnp.floor(roi[:, 1] * scale).astype(jnp.int32)
    x2 = jnp.ceil(roi[:, 2] * scale).astype(jnp.int32)
    y2 = jnp.ceil(roi[:, 3] * scale).astype(jnp.int32)
    x1 = jnp.minimum(jnp.maximum(x1, 0), fw - 1)
    y1 = jnp.minimum(jnp.maximum(y1, 0), fh - 1)
    x2 = jnp.minimum(jnp.maximum(x2, x1 + 1), fw)
    y2 = jnp.minimum(jnp.maximum(y2, y1 + 1), fh)

    ys, ye = _bin_bounds(y1, y2 - y1)
    xs, xe = _bin_bounds(x1, x2 - x1)
    leny = ye - ys
    ky = _floor_log2(leny)
    gy2 = ye - (1 << ky).astype(jnp.int32)

    def pad(a, fill):
        return jnp.concatenate(
            [a, jnp.full((_NPAD - _POST, _P), fill, jnp.int32)], axis=0)

    ky = pad(ky.astype(jnp.int32), 0)
    gy1 = pad(ys.astype(jnp.int32), 0)
    gy2 = pad(gy2.astype(jnp.int32), 0)
    xs = pad(xs.astype(jnp.int32), 0)
    xe = pad(xe.astype(jnp.int32), 1)

    grid_spec = pltpu.PrefetchScalarGridSpec(
        num_scalar_prefetch=5,
        grid=(_NPAD,),
        in_specs=[pl.BlockSpec((fh, fw, ch), lambda n, *_: (n * 0, n * 0, n * 0))],
        out_specs=pl.BlockSpec((_P * _P, 1, 1, ch), lambda n, *_: (n * 0, n, n * 0, n * 0)),
        scratch_shapes=[pltpu.VMEM((6, fh, fw, ch), jnp.float32)],
    )
    body = functools.partial(_roipool_body, fh=fh, fw=fw, ch=ch)
    return pl.pallas_call(
        body,
        grid_spec=grid_spec,
        out_shape=jax.ShapeDtypeStruct((_P * _P, _NPAD, 1, ch), jnp.float32),
    )(ky, gy1, gy2, xs, xe, featT)


def _fc1_body(p2_ref, w1_ref, b1_ref, x1_ref, acc_ref):
    k = pl.program_id(1)

    @pl.when(k == 0)
    def _():
        acc_ref[...] = jnp.zeros_like(acc_ref)

    a = p2_ref[pl.ds(k, 1)][0, :, 0, :]
    b = w1_ref[:, 0, 0, :]
    acc_ref[...] += jnp.dot(a, b, preferred_element_type=jnp.float32)

    @pl.when(k == _P * _P - 1)
    def _():
        x1_ref[0] = jnp.maximum(acc_ref[...] + b1_ref[0, 0][None, :], 0.0)


def _fc2_body(x1_ref, w2_ref, b2_ref, x2_ref, acc_ref):
    k = pl.program_id(1)

    @pl.when(k == 0)
    def _():
        acc_ref[...] = jnp.zeros_like(acc_ref)

    a = x1_ref[pl.ds(k, 1)][0]
    acc_ref[...] += jnp.dot(a, w2_ref[...], preferred_element_type=jnp.float32)

    @pl.when(k == 7)
    def _():
        x2_ref[0] = jnp.maximum(acc_ref[...] + b2_ref[0, 0][None, :], 0.0)


def _head_body(x2_ref, wd_ref, wt_ref, bd_ref, bt_ref, roi_ref,
               dec_ref, del_ref, tgt_ref):
    oh = jnp.zeros((_NPAD, 4 * _NCLS), jnp.float32)
    lg = jnp.zeros((_NPAD, _NCLS), jnp.float32)
    for k in range(8):
        a = x2_ref[k]
        oh = oh + jnp.dot(a, wd_ref[k], preferred_element_type=jnp.float32)
        lg = lg + jnp.dot(a, wt_ref[k], preferred_element_type=jnp.float32)
    oh = oh + bd_ref[0][None, :]
    lg = lg + bt_ref[0][None, :]

    m = jnp.max(lg, axis=1, keepdims=True)
    e = jnp.exp(lg - m)
    t = e / jnp.sum(e, axis=1, keepdims=True)
    tgt_ref[...] = t

    c21 = lax.broadcasted_iota(jnp.int32, (_NPAD, _NCLS), 1)
    tmax = jnp.max(t, axis=1, keepdims=True)
    mi = jnp.min(jnp.where(t == tmax, c21, _NCLS), axis=1, keepdims=True)

    c84 = lax.broadcasted_iota(jnp.int32, (_NPAD, 4 * _NCLS), 1)
    cls = c84 // 4
    sel = cls == mi
    parts = []
    for j in range(4):
        pj = jnp.sum(jnp.where(sel & (c84 % 4 == j), oh, 0.0),
                     axis=1, keepdims=True)
        parts.append(pj)
    delta = jnp.concatenate(parts, axis=1)
    del_ref[...] = delta

    x1r = roi_ref[:, 0:1]
    y1r = roi_ref[:, 1:2]
    x2r = roi_ref[:, 2:3]
    y2r = roi_ref[:, 3:4]
    wa = x2r - x1r
    ha = y2r - y1r
    cxa = x1r + 0.5 * wa
    cya = y1r + 0.5 * ha
    cx = delta[:, 0:1] * wa + cxa
    cy = delta[:, 1:2] * ha + cya
    w = jnp.exp(delta[:, 2:3]) * wa
    h = jnp.exp(delta[:, 3:4]) * ha
    dec_ref[...] = jnp.concatenate(
        [cx - 0.5 * w, cy - 0.5 * h, cx + 0.5 * w, cy + 0.5 * h], axis=1)


def _fc_head(pooled, roi_pad, W1, b1, W2, b2, Wd, bd, Wt, bt):
    ch = pooled.shape[-1]
    W1v = W1.reshape(ch, _P * _P, 1, 4096)
    x1 = pl.pallas_call(
        _fc1_body,
        grid=(8, _P * _P),
        in_specs=[
            pl.BlockSpec((_P * _P, _NPAD, 1, ch), lambda j, k: (j * 0, j * 0, j * 0, j * 0)),
            pl.BlockSpec((ch, 1, 1, 512), lambda j, k: (j * 0, k, j * 0, j)),
            pl.BlockSpec((1, 1, 512), lambda j, k: (j, j * 0, j * 0)),
        ],
        out_specs=pl.BlockSpec((1, _NPAD, 512), lambda j, k: (j, j * 0, j * 0)),
        out_shape=jax.ShapeDtypeStruct((8, _NPAD, 512), jnp.float32),
        scratch_shapes=[pltpu.VMEM((_NPAD, 512), jnp.float32)],
    )(pooled, W1v, b1.reshape(8, 1, 512))

    x2 = pl.pallas_call(
        _fc2_body,
        grid=(8, 8),
        in_specs=[
            pl.BlockSpec((8, _NPAD, 512), lambda j, k: (j * 0, j * 0, j * 0)),
            pl.BlockSpec((512, 512), lambda j, k: (k, j)),
            pl.BlockSpec((1, 1, 512), lambda j, k: (j, j * 0, j * 0)),
        ],
        out_specs=pl.BlockSpec((1, _NPAD, 512), lambda j, k: (j, j * 0, j * 0)),
        out_shape=jax.ShapeDtypeStruct((8, _NPAD, 512), jnp.float32),
        scratch_shapes=[pltpu.VMEM((_NPAD, 512), jnp.float32)],
    )(x1, W2, b2.reshape(8, 1, 512))

    dec, dlt, tgt = pl.pallas_call(
        _head_body,
        out_shape=(
            jax.ShapeDtypeStruct((_NPAD, 4), jnp.float32),
            jax.ShapeDtypeStruct((_NPAD, 4), jnp.float32),
            jax.ShapeDtypeStruct((_NPAD, _NCLS), jnp.float32),
        ),
    )(x2, Wd.reshape(8, 512, 4 * _NCLS), Wt.reshape(8, 512, _NCLS),
      bd.reshape(1, -1), bt.reshape(1, -1), roi_pad)
    return dec, dlt, tgt


def kernel(features, anchors, rpn_delta, rpn_targets, W1, b1, W2, b2,
           Wd, bd, Wt, bt, img_h, img_w):
    roi = _make_rois(anchors, rpn_delta, rpn_targets, img_h, img_w)
    roi_pad = jnp.concatenate(
        [roi, jnp.zeros((_NPAD - _POST, 4), jnp.float32)], axis=0)
    featT = jnp.transpose(features[0], (1, 2, 0))
    pooled = _roi_pool_pallas(featT, roi)
    dec, dlt, tgt = _fc_head(pooled, roi_pad, W1, b1, W2, b2, Wd, bd, Wt, bt)
    return (dec[:_POST], dlt[:_POST], tgt[:_POST])
